# baseline (speedup 1.0000x reference)
import functools

import jax
import jax.numpy as jnp
from jax import lax
from jax.experimental import pallas as pl
from jax.experimental.pallas import tpu as pltpu
from jax.experimental.pallas import tpu_sc as plsc

EMBED = 64
BATCH = 16384
NUM_CORES = 2
NUM_SUBCORES = 16
NUM_WORKERS = NUM_CORES * NUM_SUBCORES
B_PER_W = BATCH // NUM_WORKERS
L = 16
G = 32
NGRP = B_PER_W // G


@functools.partial(
    pl.kernel,
    mesh=plsc.VectorSubcoreMesh(core_axis_name="c", subcore_axis_name="s"),
    out_type=jax.ShapeDtypeStruct((BATCH, EMBED), jnp.float32),
    scratch_types=[
        pltpu.VMEM((B_PER_W,), jnp.int32),
        pltpu.VMEM((G, 8, EMBED), jnp.float32),
        pltpu.VMEM((G, 8, EMBED), jnp.float32),
        pltpu.VMEM((G, EMBED), jnp.float32),
        pltpu.VMEM((G, EMBED), jnp.float32),
        pltpu.SemaphoreType.DMA,
        pltpu.SemaphoreType.DMA,
        pltpu.SemaphoreType.DMA,
        pltpu.SemaphoreType.DMA,
    ],
)
def _tile_gather(idx_hbm, tab_hbm, out_hbm, idx_v, buf_a, buf_b, row_a,
                 row_b, sem_a, sem_b, sem_oa, sem_ob):
    wid = lax.axis_index("s") * NUM_CORES + lax.axis_index("c")
    base = wid * B_PER_W
    pltpu.sync_copy(idx_hbm.at[pl.ds(base, B_PER_W)], idx_v)

    def fire(g, buf, sem):
        for h in range(G // L):
            vec = idx_v[pl.ds(g * G + h * L, L)]
            for j in range(L):
                q = vec[j] >> 3
                pltpu.async_copy(tab_hbm.at[q], buf.at[h * L + j], sem)

    def out_slab(g):
        return out_hbm.at[pl.ds(base + g * G, G), :]

    def drain_extract(g, buf, sem, row, sem_o):
        for j in range(G):
            pltpu.make_async_copy(tab_hbm.at[0], buf.at[j], sem).wait()

        @pl.when(g >= 2)
        def _():
            pltpu.make_async_copy(row, out_slab(g - 2), sem_o).wait()

        for h in range(G // L):
            vec = idx_v[pl.ds(g * G + h * L, L)]
            for j in range(L):
                r = vec[j] & 7
                for k in range(EMBED // L):
                    row[h * L + j, pl.ds(k * L, L)] = buf[h * L + j, r,
                                                          pl.ds(k * L, L)]
        pltpu.async_copy(row, out_slab(g), sem_o)

    fire(0, buf_a, sem_a)
    fire(1, buf_b, sem_b)

    def pair(i, carry):
        g = i * 2
        drain_extract(g, buf_a, sem_a, row_a, sem_oa)

        @pl.when(g + 2 < NGRP)
        def _():
            fire(g + 2, buf_a, sem_a)

        drain_extract(g + 1, buf_b, sem_b, row_b, sem_ob)

        @pl.when(g + 3 < NGRP)
        def _():
            fire(g + 3, buf_b, sem_b)

        return carry

    lax.fori_loop(0, NGRP // 2, pair, 0)
    pltpu.make_async_copy(row_a, out_slab(NGRP - 2), sem_oa).wait()
    pltpu.make_async_copy(row_b, out_slab(NGRP - 1), sem_ob).wait()


def kernel(x, table):
    idx = x[:, -1].astype(jnp.int32)
    return _tile_gather(idx, table.reshape(125000, 8, EMBED))
